# Initial kernel scaffold; baseline (speedup 1.0000x reference)
import functools

import jax
import jax.numpy as jnp
from jax import lax
from jax.experimental import pallas as pl
from jax.experimental.pallas import tpu as pltpu
from jax.experimental.pallas import tpu_sc as plsc

NN = 10000
NE = 320000
DF = 128
HF = 128
CF = 16

NC = 2
NS = 16
LANES = 16

CH = 80
KH = NE // NS // CH
CM = 80
KM = NE // NC // NS // CM
RPS = NN // NS

_mesh = plsc.VectorSubcoreMesh(core_axis_name="c", subcore_axis_name="s")


def _fill(ref, nrows, ncols, value):
    @pl.loop(0, nrows)
    def _(i):
        @pl.loop(0, ncols, step=LANES)
        def _(j):
            ref[i, pl.ds(j, LANES)] = jnp.full((LANES,), value, ref.dtype)


def _hist_body(idx_hbm, deg_hbm, idx_v, ones_v, zbuf_v, acc_sh):
    c = lax.axis_index("c")
    s = lax.axis_index("s")
    _fill(zbuf_v, RPS, 16, 0.0)
    _fill(ones_v, CH, 16, 1.0)
    pltpu.sync_copy(zbuf_v, acc_sh.at[pl.ds(s * RPS, RPS)])
    plsc.subcore_barrier()
    pltpu.sync_copy(idx_hbm.at[c, s], idx_v)

    @pl.loop(0, KH)
    def _(j):
        pltpu.sync_copy(ones_v, acc_sh.at[idx_v.at[j]], add=True)

    plsc.subcore_barrier()
    pltpu.sync_copy(acc_sh.at[pl.ds(s * RPS, RPS)],
                    deg_hbm.at[c, pl.ds(s * RPS, RPS)])


def _msg_body(em_hbm, h_hbm, out_hbm, sidx_v, didx_v, rows_v, zbuf_v, acc_sh):
    c = lax.axis_index("c")
    s = lax.axis_index("s")
    _fill(zbuf_v, RPS // 5, DF, 0.0)

    @pl.loop(0, 5)
    def _(t):
        pltpu.sync_copy(zbuf_v, acc_sh.at[pl.ds(s * RPS + t * (RPS // 5),
                                                RPS // 5)])

    plsc.subcore_barrier()
    pltpu.sync_copy(em_hbm.at[0, c, s], sidx_v)
    pltpu.sync_copy(em_hbm.at[1, c, s], didx_v)

    @pl.loop(0, KM)
    def _(j):
        pltpu.sync_copy(h_hbm.at[sidx_v.at[j]], rows_v)
        pltpu.sync_copy(rows_v, acc_sh.at[didx_v.at[j]], add=True)

    plsc.subcore_barrier()
    pltpu.sync_copy(acc_sh.at[pl.ds(s * RPS, RPS)],
                    out_hbm.at[c, pl.ds(s * RPS, RPS)])


@jax.jit
def _sc_hist(eh):
    return pl.kernel(
        _hist_body,
        out_type=jax.ShapeDtypeStruct((2, NN, 16), jnp.float32),
        mesh=_mesh,
        scratch_types=[
            pltpu.VMEM((KH, CH), jnp.int32),
            pltpu.VMEM((CH, 16), jnp.float32),
            pltpu.VMEM((RPS, 16), jnp.float32),
            pltpu.VMEM_SHARED((NN, 16), jnp.float32),
        ],
    )(eh)


@jax.jit
def _sc_msg(em, h):
    return pl.kernel(
        _msg_body,
        out_type=jax.ShapeDtypeStruct((NC, NN, DF), jnp.float32),
        mesh=_mesh,
        scratch_types=[
            pltpu.VMEM((KM, CM), jnp.int32),
            pltpu.VMEM((KM, CM), jnp.int32),
            pltpu.VMEM((CM, DF), jnp.float32),
            pltpu.VMEM((RPS // 5, DF), jnp.float32),
            pltpu.VMEM_SHARED((NN, DF), jnp.float32),
        ],
    )(em, h)


def _norm_col(deg_block):
    d = deg_block[:, 0:1]
    return jnp.where(d > 0, lax.rsqrt(jnp.maximum(d, 1e-12)), 0.0)


def _scale_body(x_ref, deg_ref, h_ref):
    h_ref[...] = x_ref[...] * _norm_col(deg_ref[0])


def _final_body(p_ref, deg_ref, w1_ref, b1_ref, wfc_ref, bfc_ref, o_ref):
    agg = (p_ref[0] + p_ref[1]) * _norm_col(deg_ref[0])
    z = jnp.dot(agg, w1_ref[...], preferred_element_type=jnp.float32)
    z = jnp.maximum(z + b1_ref[...], 0.0)
    o = jnp.dot(z, wfc_ref[...], preferred_element_type=jnp.float32)
    o_ref[...] = jax.nn.sigmoid(o + bfc_ref[...])


BN = 1000


@jax.jit
def _tc_scale(x, deg):
    return pl.pallas_call(
        _scale_body,
        grid=(NN // BN,),
        in_specs=[
            pl.BlockSpec((BN, DF), lambda i: (i, 0)),
            pl.BlockSpec((1, BN, 16), lambda i: (0, i, 0)),
        ],
        out_specs=pl.BlockSpec((BN, DF), lambda i: (i, 0)),
        out_shape=jax.ShapeDtypeStruct((NN, DF), jnp.float32),
    )(x, deg)


@jax.jit
def _tc_final(parts, deg, W1, b1, Wfc, bfc):
    return pl.pallas_call(
        _final_body,
        grid=(NN // BN,),
        in_specs=[
            pl.BlockSpec((NC, BN, DF), lambda i: (0, i, 0)),
            pl.BlockSpec((1, BN, 16), lambda i: (1, i, 0)),
            pl.BlockSpec((DF, HF), lambda i: (0, 0)),
            pl.BlockSpec((1, HF), lambda i: (0, 0)),
            pl.BlockSpec((HF, CF), lambda i: (0, 0)),
            pl.BlockSpec((1, CF), lambda i: (0, 0)),
        ],
        out_specs=pl.BlockSpec((BN, CF), lambda i: (i, 0)),
        out_shape=jax.ShapeDtypeStruct((NN, CF), jnp.float32),
    )(parts, deg, W1, b1, Wfc, bfc)


def kernel(edge_index, in_feat, W1, b1, Wfc, bfc):
    eh = edge_index.reshape(2, NS, KH, CH)
    em = edge_index.reshape(2, NC, NS, KM, CM)
    deg = _sc_hist(eh)
    h = _tc_scale(in_feat, deg)
    parts = _sc_msg(em, h)
    return _tc_final(parts, deg, W1, b1.reshape(1, HF), Wfc, bfc.reshape(1, CF))


# speedup vs baseline: 8.4295x; 8.4295x over previous
import dataclasses
import functools

import jax
import jax.numpy as jnp
from jax import lax
from jax.experimental import pallas as pl
from jax.experimental.pallas import tpu as pltpu
from jax.experimental.pallas import tpu_sc as plsc

NN = 10000
NE = 320000
DF = 128
HF = 128
CF = 16

NC = 2
NS = 16
LANES = 16

CW = 128
KH3 = 157
EHP = NS * KH3 * CW
PADH = EHP - NE
CM = 80
KM = NE // NC // NS // CM
NP = 10240
RPS = NP // NS

_mesh = plsc.VectorSubcoreMesh(core_axis_name="c", subcore_axis_name="s")


def _hist_body(idx_hbm, zeros_hbm, deg_hbm, idx_v, cnt_v, part_v, acc_v,
               out16_v, sh_sh):
    c = lax.axis_index("c")
    s = lax.axis_index("s")
    pltpu.sync_copy(zeros_hbm, cnt_v)
    pltpu.sync_copy(idx_hbm.at[c, s], idx_v)
    ones16 = jnp.full((LANES,), 1.0, jnp.float32)

    @pl.loop(0, KH3)
    def _(r):
        @pl.loop(0, CW // LANES)
        def _(k):
            iv = idx_v[r, pl.ds(k * LANES, LANES)]
            plsc.addupdate_scatter(cnt_v, [iv], ones16)

    pltpu.sync_copy(cnt_v, sh_sh.at[s])
    plsc.subcore_barrier()

    pltpu.sync_copy(zeros_hbm.at[pl.ds(0, RPS)], acc_v)

    @pl.loop(0, NS)
    def _(k):
        pltpu.sync_copy(sh_sh.at[k, pl.ds(s * RPS, RPS)], part_v)

        @pl.loop(0, RPS // LANES)
        def _(g):
            sl = pl.ds(g * LANES, LANES)
            acc_v[sl] = acc_v[sl] + part_v[sl]

    iot = lax.iota(jnp.int32, LANES)
    HH = RPS // 2

    @pl.loop(0, 2)
    def _(hh):
        @pl.loop(0, HH // LANES)
        def _(g):
            v = acc_v[pl.ds(hh * HH + g * LANES, LANES)]

            @pl.loop(0, LANES)
            def _(j):
                plsc.store_scatter(
                    out16_v,
                    [iot + g * LANES, jnp.full((LANES,), j, jnp.int32)], v)

        pltpu.sync_copy(out16_v,
                        deg_hbm.at[c, pl.ds(s * RPS + hh * HH, HH)])


def _msg_body(em_hbm, h_hbm, zeros_hbm, out_hbm, sidx_v, didx_v, rows_v,
              acc_sh):
    c = lax.axis_index("c")
    s = lax.axis_index("s")
    pltpu.sync_copy(zeros_hbm.at[pl.ds(s * RPS, RPS)],
                    acc_sh.at[pl.ds(s * RPS, RPS)])
    plsc.subcore_barrier()
    pltpu.sync_copy(em_hbm.at[0, c, s], sidx_v)
    pltpu.sync_copy(em_hbm.at[1, c, s], didx_v)

    @pl.loop(0, KM)
    def _(j):
        pltpu.sync_copy(h_hbm.at[sidx_v.at[j]], rows_v)
        pltpu.sync_copy(rows_v, acc_sh.at[didx_v.at[j]], add=True)

    plsc.subcore_barrier()
    pltpu.sync_copy(acc_sh.at[pl.ds(s * RPS, RPS)],
                    out_hbm.at[c, pl.ds(s * RPS, RPS)])


_cp = pltpu.CompilerParams()
if "needs_layout_passes" in pltpu.CompilerParams.__dataclass_fields__:
    _cp = dataclasses.replace(_cp, needs_layout_passes=False)


@jax.jit
def _sc_hist(eh, zeros1):
    return pl.kernel(
        _hist_body,
        out_type=jax.ShapeDtypeStruct((2, NP, 16), jnp.float32),
        mesh=_mesh,
        compiler_params=_cp,
        scratch_types=[
            pltpu.VMEM((KH3, CW), jnp.int32),
            pltpu.VMEM((NP,), jnp.float32),
            pltpu.VMEM((RPS,), jnp.float32),
            pltpu.VMEM((RPS,), jnp.float32),
            pltpu.VMEM((RPS // 2, 16), jnp.float32),
            pltpu.VMEM_SHARED((NS, NP), jnp.float32),
        ],
    )(eh, zeros1)


@jax.jit
def _sc_msg(em, h, zeros128):
    return pl.kernel(
        _msg_body,
        out_type=jax.ShapeDtypeStruct((NC, NP, DF), jnp.float32),
        mesh=_mesh,
        scratch_types=[
            pltpu.VMEM((KM, CM), jnp.int32),
            pltpu.VMEM((KM, CM), jnp.int32),
            pltpu.VMEM((CM, DF), jnp.float32),
            pltpu.VMEM_SHARED((NP, DF), jnp.float32),
        ],
    )(em, h, zeros128)


def _norm_col(deg_block):
    d = deg_block[:, 0:1]
    return jnp.where(d > 0, lax.rsqrt(jnp.maximum(d, 1e-12)), 0.0)


def _scale_body(x_ref, deg_ref, h_ref):
    h_ref[...] = x_ref[...] * _norm_col(deg_ref[0])


def _final_body(p_ref, deg_ref, w1_ref, b1_ref, wfc_ref, bfc_ref, o_ref):
    agg = (p_ref[0] + p_ref[1]) * _norm_col(deg_ref[0])
    z = jnp.dot(agg, w1_ref[...], preferred_element_type=jnp.float32)
    z = jnp.maximum(z + b1_ref[...], 0.0)
    o = jnp.dot(z, wfc_ref[...], preferred_element_type=jnp.float32)
    o_ref[...] = jax.nn.sigmoid(o + bfc_ref[...])


BN = 1000


@jax.jit
def _tc_scale(x, deg):
    return pl.pallas_call(
        _scale_body,
        grid=(NN // BN,),
        in_specs=[
            pl.BlockSpec((BN, DF), lambda i: (i, 0)),
            pl.BlockSpec((1, BN, 16), lambda i: (0, i, 0)),
        ],
        out_specs=pl.BlockSpec((BN, DF), lambda i: (i, 0)),
        out_shape=jax.ShapeDtypeStruct((NN, DF), jnp.float32),
    )(x, deg)


@jax.jit
def _tc_final(parts, deg, W1, b1, Wfc, bfc):
    return pl.pallas_call(
        _final_body,
        grid=(NN // BN,),
        in_specs=[
            pl.BlockSpec((NC, BN, DF), lambda i: (0, i, 0)),
            pl.BlockSpec((1, BN, 16), lambda i: (1, i, 0)),
            pl.BlockSpec((DF, HF), lambda i: (0, 0)),
            pl.BlockSpec((1, HF), lambda i: (0, 0)),
            pl.BlockSpec((HF, CF), lambda i: (0, 0)),
            pl.BlockSpec((1, CF), lambda i: (0, 0)),
        ],
        out_specs=pl.BlockSpec((BN, CF), lambda i: (i, 0)),
        out_shape=jax.ShapeDtypeStruct((NN, CF), jnp.float32),
    )(parts, deg, W1, b1, Wfc, bfc)


def kernel(edge_index, in_feat, W1, b1, Wfc, bfc):
    pad = jnp.broadcast_to(
        jnp.arange(PADH, dtype=jnp.int32) % (NP - NN) + NN, (2, PADH))
    eh = jnp.concatenate([edge_index, pad], axis=1).reshape(2, NS, KH3, CW)
    em = edge_index.reshape(2, NC, NS, KM, CM)
    deg = _sc_hist(eh, jnp.zeros((NP,), jnp.float32))
    h = _tc_scale(in_feat, deg)
    parts = _sc_msg(em, h, jnp.zeros((NP, DF), jnp.float32))
    return _tc_final(parts, deg, W1, b1.reshape(1, HF), Wfc, bfc.reshape(1, CF))


# speedup vs baseline: 11.4224x; 1.3551x over previous
import dataclasses
import functools

import jax
import jax.numpy as jnp
from jax import lax
from jax.experimental import pallas as pl
from jax.experimental.pallas import tpu as pltpu
from jax.experimental.pallas import tpu_sc as plsc

NN = 10000
NE = 320000
DF = 128
HF = 128
CF = 16

NC = 2
NS = 16
LANES = 16

CW = 128
KH3 = 157
EHP = NS * KH3 * CW
PADH = EHP - NE
CM = 128
GT = 8
NG = 10
KM = NG * GT
EMP = NC * NS * KM * CM
PADM = EMP - NE
NP = 10240
RPS = NP // NS

_mesh = plsc.VectorSubcoreMesh(core_axis_name="c", subcore_axis_name="s")


def _hist_body(idx_hbm, zeros_hbm, deg_hbm, idx_v, cnt_v, part_v, acc_v,
               out16_v, sh_sh):
    c = lax.axis_index("c")
    s = lax.axis_index("s")
    pltpu.sync_copy(zeros_hbm, cnt_v)
    pltpu.sync_copy(idx_hbm.at[c, s], idx_v)
    ones16 = jnp.full((LANES,), 1.0, jnp.float32)

    @pl.loop(0, KH3)
    def _(r):
        @pl.loop(0, CW // LANES)
        def _(k):
            iv = idx_v[r, pl.ds(k * LANES, LANES)]
            plsc.addupdate_scatter(cnt_v, [iv], ones16)

    pltpu.sync_copy(cnt_v, sh_sh.at[s])
    plsc.subcore_barrier()

    pltpu.sync_copy(zeros_hbm.at[pl.ds(0, RPS)], acc_v)

    @pl.loop(0, NS)
    def _(k):
        pltpu.sync_copy(sh_sh.at[k, pl.ds(s * RPS, RPS)], part_v)

        @pl.loop(0, RPS // LANES)
        def _(g):
            sl = pl.ds(g * LANES, LANES)
            acc_v[sl] = acc_v[sl] + part_v[sl]

    iot = lax.iota(jnp.int32, LANES)
    HH = RPS // 2

    @pl.loop(0, 2)
    def _(hh):
        @pl.loop(0, HH // LANES)
        def _(g):
            v = acc_v[pl.ds(hh * HH + g * LANES, LANES)]

            @pl.loop(0, LANES)
            def _(j):
                plsc.store_scatter(
                    out16_v,
                    [iot + g * LANES, jnp.full((LANES,), j, jnp.int32)], v)

        pltpu.sync_copy(out16_v,
                        deg_hbm.at[c, pl.ds(s * RPS + hh * HH, HH)])


def _msg_body(em_hbm, h_hbm, zeros_hbm, out_hbm, st_v, dt_v, rows_v,
              acc_sh, gsem, isem):
    c = lax.axis_index("c")
    s = lax.axis_index("s")
    pltpu.sync_copy(zeros_hbm.at[pl.ds(s * RPS, RPS)],
                    acc_sh.at[pl.ds(s * RPS, RPS)])
    plsc.subcore_barrier()
    pltpu.sync_copy(em_hbm.at[0, c, s, 0], st_v.at[0])
    pltpu.sync_copy(em_hbm.at[1, c, s, 0], dt_v.at[0])
    pltpu.async_copy(em_hbm.at[0, c, s, 1], st_v.at[1], isem)
    pltpu.async_copy(em_hbm.at[1, c, s, 1], dt_v.at[1], isem)
    pltpu.async_copy(h_hbm.at[st_v.at[0, 0]], rows_v.at[0], gsem)

    @pl.loop(0, KM)
    def _(j):
        g = j // GT
        t = j - g * GT
        p = j % 2
        q = g % 2
        pltpu.make_async_copy(h_hbm.at[st_v.at[0, 0]], rows_v.at[p],
                              gsem).wait()
        nx = j + 1

        @pl.when((nx % GT == 0) & (nx < KM))
        def _():
            pltpu.make_async_copy(em_hbm.at[0, c, s, 0], st_v.at[0],
                                  isem).wait()
            pltpu.make_async_copy(em_hbm.at[1, c, s, 0], dt_v.at[0],
                                  isem).wait()

        @pl.when(nx < KM)
        def _():
            pltpu.async_copy(
                h_hbm.at[st_v.at[(nx // GT) % 2, nx % GT]],
                rows_v.at[nx % 2], gsem)

        pltpu.sync_copy(rows_v.at[p], acc_sh.at[dt_v.at[q, t]], add=True)

        @pl.when((t == 0) & (g >= 1) & (g + 1 < NG))
        def _():
            pltpu.async_copy(em_hbm.at[0, c, s, g + 1], st_v.at[1 - q], isem)
            pltpu.async_copy(em_hbm.at[1, c, s, g + 1], dt_v.at[1 - q], isem)

    plsc.subcore_barrier()
    pltpu.sync_copy(acc_sh.at[pl.ds(s * RPS, RPS)],
                    out_hbm.at[c, pl.ds(s * RPS, RPS)])


_cp = pltpu.CompilerParams()
if "needs_layout_passes" in pltpu.CompilerParams.__dataclass_fields__:
    _cp = dataclasses.replace(_cp, needs_layout_passes=False)


@jax.jit
def _sc_hist(eh, zeros1):
    return pl.kernel(
        _hist_body,
        out_type=jax.ShapeDtypeStruct((2, NP, 16), jnp.float32),
        mesh=_mesh,
        compiler_params=_cp,
        scratch_types=[
            pltpu.VMEM((KH3, CW), jnp.int32),
            pltpu.VMEM((NP,), jnp.float32),
            pltpu.VMEM((RPS,), jnp.float32),
            pltpu.VMEM((RPS,), jnp.float32),
            pltpu.VMEM((RPS // 2, 16), jnp.float32),
            pltpu.VMEM_SHARED((NS, NP), jnp.float32),
        ],
    )(eh, zeros1)


@jax.jit
def _sc_msg(em, h, zeros128):
    return pl.kernel(
        _msg_body,
        out_type=jax.ShapeDtypeStruct((NC, NP, DF), jnp.float32),
        mesh=_mesh,
        scratch_types=[
            pltpu.VMEM((2, GT, CM), jnp.int32),
            pltpu.VMEM((2, GT, CM), jnp.int32),
            pltpu.VMEM((2, CM, DF), jnp.float32),
            pltpu.VMEM_SHARED((NP, DF), jnp.float32),
            pltpu.SemaphoreType.DMA,
            pltpu.SemaphoreType.DMA,
        ],
    )(em, h, zeros128)


def _norm_col(deg_block):
    d = deg_block[:, 0:1]
    return jnp.where(d > 0, lax.rsqrt(jnp.maximum(d, 1e-12)), 0.0)


def _scale_body(x_ref, deg_ref, h_ref):
    h_ref[...] = x_ref[...] * _norm_col(deg_ref[0])


def _final_body(p_ref, deg_ref, w1_ref, b1_ref, wfc_ref, bfc_ref, o_ref):
    agg = (p_ref[0] + p_ref[1]) * _norm_col(deg_ref[0])
    z = jnp.dot(agg, w1_ref[...], preferred_element_type=jnp.float32)
    z = jnp.maximum(z + b1_ref[...], 0.0)
    o = jnp.dot(z, wfc_ref[...], preferred_element_type=jnp.float32)
    o_ref[...] = jax.nn.sigmoid(o + bfc_ref[...])


BN = 1000


@jax.jit
def _tc_scale(x, deg):
    return pl.pallas_call(
        _scale_body,
        grid=(NN // BN,),
        in_specs=[
            pl.BlockSpec((BN, DF), lambda i: (i, 0)),
            pl.BlockSpec((1, BN, 16), lambda i: (0, i, 0)),
        ],
        out_specs=pl.BlockSpec((BN, DF), lambda i: (i, 0)),
        out_shape=jax.ShapeDtypeStruct((NN, DF), jnp.float32),
    )(x, deg)


@jax.jit
def _tc_final(parts, deg, W1, b1, Wfc, bfc):
    return pl.pallas_call(
        _final_body,
        grid=(NN // BN,),
        in_specs=[
            pl.BlockSpec((NC, BN, DF), lambda i: (0, i, 0)),
            pl.BlockSpec((1, BN, 16), lambda i: (1, i, 0)),
            pl.BlockSpec((DF, HF), lambda i: (0, 0)),
            pl.BlockSpec((1, HF), lambda i: (0, 0)),
            pl.BlockSpec((HF, CF), lambda i: (0, 0)),
            pl.BlockSpec((1, CF), lambda i: (0, 0)),
        ],
        out_specs=pl.BlockSpec((BN, CF), lambda i: (i, 0)),
        out_shape=jax.ShapeDtypeStruct((NN, CF), jnp.float32),
    )(parts, deg, W1, b1, Wfc, bfc)


def kernel(edge_index, in_feat, W1, b1, Wfc, bfc):
    pad = jnp.broadcast_to(
        jnp.arange(PADH, dtype=jnp.int32) % (NP - NN) + NN, (2, PADH))
    eh = jnp.concatenate([edge_index, pad], axis=1).reshape(2, NS, KH3, CW)
    r = jnp.arange(PADM, dtype=jnp.int32)
    mpad = jnp.stack([r % NN, r % (NP - NN) + NN])
    em = jnp.concatenate([edge_index, mpad],
                         axis=1).reshape(2, NC, NS, NG, GT, CM)
    deg = _sc_hist(eh, jnp.zeros((NP,), jnp.float32))
    h = _tc_scale(in_feat, deg)
    parts = _sc_msg(em, h, jnp.zeros((NP, DF), jnp.float32))
    return _tc_final(parts, deg, W1, b1.reshape(1, HF), Wfc, bfc.reshape(1, CF))
